# Initial kernel scaffold; baseline (speedup 1.0000x reference)
import jax
import jax.numpy as jnp
from jax.experimental import pallas as pl


def kernel(hidden_states, gate_w, w1, w2, w3):
    raise NotImplementedError("write your pallas kernel here")


# speedup vs baseline: 1.1271x; 1.1271x over previous
import functools

import jax
import jax.numpy as jnp
from jax import lax
from jax.experimental import pallas as pl
from jax.experimental.pallas import tpu as pltpu
from jax.experimental.pallas import tpu_sc as plsc

JITTER_EPS = 0.01


def _router_body(x_ref, gwt_ref, i1_ref, i2_ref, m1_ref, m2_ref):
    x = x_ref[...]
    s = jnp.dot(x, gwt_ref[...], preferred_element_type=jnp.float32)
    E = s.shape[-1]
    neg_inf = jnp.float32(-jnp.inf)
    m1 = jnp.max(s, axis=-1, keepdims=True)
    i1 = jnp.argmax(s, axis=-1).astype(jnp.int32)
    f1 = jnp.maximum(jnp.abs(s), m1)
    l1 = jnp.where((m1 - s) / f1 > 2.0 * JITTER_EPS, neg_inf, s)
    mult1 = 1.0 / jnp.sum(jnp.exp(l1 - m1), axis=-1)
    cols = lax.broadcasted_iota(jnp.int32, s.shape, 1)
    s2 = jnp.where(cols == i1[:, None], neg_inf, s)
    m2 = jnp.max(s2, axis=-1, keepdims=True)
    i2 = jnp.argmax(s2, axis=-1).astype(jnp.int32)
    f2 = jnp.maximum(jnp.abs(s), m2)
    l2 = jnp.where((m2 - s) / f2 > 2.0 * JITTER_EPS, neg_inf, s2)
    mult2 = 1.0 / jnp.sum(jnp.exp(l2 - m2), axis=-1)
    i1_ref[...] = i1
    i2_ref[...] = i2
    m1_ref[...] = mult1
    m2_ref[...] = mult2


def _run_router(x, gate_w, T, H, E, RBT):
    grid = (T // RBT,)
    return pl.pallas_call(
        _router_body,
        grid=grid,
        in_specs=[
            pl.BlockSpec((RBT, H), lambda b: (b, 0)),
            pl.BlockSpec((H, E), lambda b: (0, 0)),
        ],
        out_specs=[
            pl.BlockSpec((RBT,), lambda b: (b,)),
            pl.BlockSpec((RBT,), lambda b: (b,)),
            pl.BlockSpec((RBT,), lambda b: (b,)),
            pl.BlockSpec((RBT,), lambda b: (b,)),
        ],
        out_shape=[
            jax.ShapeDtypeStruct((T,), jnp.int32),
            jax.ShapeDtypeStruct((T,), jnp.int32),
            jax.ShapeDtypeStruct((T,), jnp.float32),
            jax.ShapeDtypeStruct((T,), jnp.float32),
        ],
    )(x, gate_w.T)


def _dispatch_metadata(i1, i2, mult1, mult2, T, E, BT, NB, P):
    e_flat = jnp.concatenate([i1, i2])
    w_flat = jnp.concatenate([mult1, mult2])
    order = jnp.argsort(e_flat)
    e_sorted = e_flat[order]
    counts = jnp.bincount(e_flat, length=E).astype(jnp.int32)
    csum = jnp.cumsum(counts)
    offsets = csum - counts
    padded_counts = ((counts + BT - 1) // BT) * BT
    padded_cum = jnp.cumsum(padded_counts)
    padded_offsets = padded_cum - padded_counts
    j = jnp.arange(2 * T, dtype=jnp.int32)
    pad_pos = padded_offsets[e_sorted] + (j - offsets[e_sorted])
    tok_sorted = (order % T).astype(jnp.int32)
    row_tok = jnp.zeros((P,), jnp.int32).at[pad_pos].set(tok_sorted)
    w_row = jnp.zeros((P,), jnp.float32).at[pad_pos].set(w_flat[order])
    inv_pos = jnp.zeros((2 * T,), jnp.int32).at[order].set(pad_pos)
    block_starts = jnp.arange(NB, dtype=jnp.int32) * BT
    total_padded = padded_cum[-1]
    valid = (block_starts < total_padded).astype(jnp.int32)
    be = jnp.searchsorted(padded_cum, block_starts, side='right').astype(jnp.int32)
    nvalid = total_padded // BT
    be_last = be[nvalid - 1]
    be = jnp.where(valid == 1, be, be_last)
    return row_tok, w_row, inv_pos[:T], inv_pos[T:], be, valid


def _make_gather(P, H):
    info = plsc.get_sparse_core_info()
    NC, NS = info.num_cores, info.num_subcores
    NW = NC * NS
    pw = P // NW
    CH = 64
    assert pw % CH == 0
    mesh = plsc.VectorSubcoreMesh(core_axis_name="c", subcore_axis_name="s")

    @functools.partial(
        pl.kernel, mesh=mesh,
        out_type=jax.ShapeDtypeStruct((P, H), jnp.float32),
        scratch_types=[
            pltpu.VMEM((pw,), jnp.int32),
            pltpu.VMEM((CH, H), jnp.float32),
            pltpu.SemaphoreType.DMA,
        ],
    )
    def gk(x_hbm, idx_hbm, out_hbm, idx_v, rows_v, sem):
        wid = lax.axis_index("s") * NC + lax.axis_index("c")
        base = wid * pw
        pltpu.sync_copy(idx_hbm.at[pl.ds(base, pw)], idx_v)
        for ch in range(pw // CH):
            pltpu.async_copy(
                x_hbm.at[idx_v.at[pl.ds(ch * CH, CH)]], rows_v, sem).wait()
            pltpu.sync_copy(rows_v, out_hbm.at[pl.ds(base + ch * CH, CH)])

    return gk


def _make_ffn(P, H, F, BT, FT, NB, NF):
    def body(be_ref, valid_ref, xs_ref, w1_ref, w3_ref, w2_ref, wr_ref, out_ref):
        b = pl.program_id(0)
        f = pl.program_id(1)

        @pl.when(valid_ref[b] == 1)
        def _():
            xs = xs_ref[...]
            a = jnp.dot(xs, w1_ref[0], preferred_element_type=jnp.float32)
            c = jnp.dot(xs, w3_ref[0], preferred_element_type=jnp.float32)
            h = (a * jax.nn.sigmoid(a)) * c
            y = jnp.dot(h, w2_ref[0], preferred_element_type=jnp.float32)

            @pl.when(f == 0)
            def _():
                out_ref[...] = y

            @pl.when(f != 0)
            def _():
                out_ref[...] += y

            @pl.when(f == NF - 1)
            def _():
                out_ref[...] *= wr_ref[...]

        @pl.when((valid_ref[b] == 0) & (f == NF - 1))
        def _():
            out_ref[...] = jnp.zeros_like(out_ref)

    grid_spec = pltpu.PrefetchScalarGridSpec(
        num_scalar_prefetch=2,
        grid=(NB, NF),
        in_specs=[
            pl.BlockSpec((BT, H), lambda b, f, be, va: (b, 0)),
            pl.BlockSpec((1, H, FT), lambda b, f, be, va: (be[b], 0, f)),
            pl.BlockSpec((1, H, FT), lambda b, f, be, va: (be[b], 0, f)),
            pl.BlockSpec((1, FT, H), lambda b, f, be, va: (be[b], f, 0)),
            pl.BlockSpec((BT, 1), lambda b, f, be, va: (b, 0)),
        ],
        out_specs=pl.BlockSpec((BT, H), lambda b, f, be, va: (b, 0)),
    )
    return pl.pallas_call(
        body,
        grid_spec=grid_spec,
        out_shape=jax.ShapeDtypeStruct((P, H), jnp.float32),
        compiler_params=pltpu.CompilerParams(
            dimension_semantics=("arbitrary", "arbitrary")),
    )


def _make_combine(P, H, T):
    info = plsc.get_sparse_core_info()
    NC, NS = info.num_cores, info.num_subcores
    NW = NC * NS
    tw = T // NW
    CH = 32
    assert tw % CH == 0
    nlane = H // 16
    mesh = plsc.VectorSubcoreMesh(core_axis_name="c", subcore_axis_name="s")

    @functools.partial(
        pl.kernel, mesh=mesh,
        out_type=jax.ShapeDtypeStruct((T, H), jnp.float32),
        scratch_types=[
            pltpu.VMEM((tw,), jnp.int32),
            pltpu.VMEM((tw,), jnp.int32),
            pltpu.VMEM((CH, H), jnp.float32),
            pltpu.VMEM((CH, H), jnp.float32),
            pltpu.SemaphoreType.DMA,
            pltpu.SemaphoreType.DMA,
        ],
    )
    def ck(y_hbm, inv0_hbm, inv1_hbm, out_hbm, i0_v, i1_v, bufa, bufb, sema, semb):
        wid = lax.axis_index("s") * NC + lax.axis_index("c")
        base = wid * tw
        pltpu.sync_copy(inv0_hbm.at[pl.ds(base, tw)], i0_v)
        pltpu.sync_copy(inv1_hbm.at[pl.ds(base, tw)], i1_v)
        for ch in range(tw // CH):
            ca = pltpu.async_copy(
                y_hbm.at[i0_v.at[pl.ds(ch * CH, CH)]], bufa, sema)
            cb = pltpu.async_copy(
                y_hbm.at[i1_v.at[pl.ds(ch * CH, CH)]], bufb, semb)
            ca.wait()
            cb.wait()

            def row_body(r, carry):
                def lane_body(c, carry2):
                    sl = pl.ds(c * 16, 16)
                    bufa[r, sl] = bufa[r, sl] + bufb[r, sl]
                    return carry2
                return lax.fori_loop(0, nlane, lane_body, carry, unroll=8)

            lax.fori_loop(0, CH, row_body, 0)
            pltpu.sync_copy(bufa, out_hbm.at[pl.ds(base + ch * CH, CH)])

    return ck


def kernel(hidden_states, gate_w, w1, w2, w3):
    B, S, H = hidden_states.shape
    E, _, F = w1.shape
    T = B * S
    x = hidden_states.reshape(T, H)

    BT = 256
    FT = 512
    NB = (2 * T) // BT + E
    P = NB * BT
    NF = F // FT
    RBT = 256

    i1, i2, mult1, mult2 = _run_router(x, gate_w, T, H, E, RBT)
    row_tok, w_row, inv0, inv1, be, valid = _dispatch_metadata(
        i1, i2, mult1, mult2, T, E, BT, NB, P)

    x_sorted = _make_gather(P, H)(x, row_tok)
    y_sorted = _make_ffn(P, H, F, BT, FT, NB, NF)(
        be, valid, x_sorted, w1, w3, w2, w_row.reshape(P, 1))
    out = _make_combine(P, H, T)(y_sorted, inv0, inv1)
    return out.reshape(B, S, H)
